# Initial kernel scaffold; baseline (speedup 1.0000x reference)
import functools

import jax
import jax.numpy as jnp
import numpy as np
from jax.experimental import pallas as pl
from jax.experimental.pallas import tpu as pltpu

N_AB = 5000
N_AG = 5000
N = N_AB + N_AG
E = 40000
E4 = 4 * E
D = 256
H = 8
DH = D // H
BLOCKS = 3
GAT_LAYERS = 2


def _ln(x, s, b):
    mu = jnp.mean(x, axis=-1, keepdims=True)
    var = jnp.var(x, axis=-1, keepdims=True)
    return (x - mu) / jnp.sqrt(var + 1e-5) * s + b


def _seg_softmax(scores, seg, n):
    m = jax.ops.segment_max(scores, seg, num_segments=n)
    m = jnp.where(jnp.isfinite(m), m, 0.0)
    ex = jnp.exp(scores - m[seg])
    den = jax.ops.segment_sum(ex, seg, num_segments=n)
    return ex / (den[seg] + 1e-9)


def _relu_add_pallas(a, b, bias):
    def body(a_ref, b_ref, bias_ref, o_ref):
        o_ref[...] = jnp.maximum(a_ref[...] + b_ref[...] + bias_ref[...], 0.0)

    grid = (a.shape[0] // 2000,)
    return pl.pallas_call(
        body,
        grid=grid,
        in_specs=[
            pl.BlockSpec((2000, D), lambda i: (i, 0)),
            pl.BlockSpec((2000, D), lambda i: (i, 0)),
            pl.BlockSpec((1, D), lambda i: (0, 0)),
        ],
        out_specs=pl.BlockSpec((2000, D), lambda i: (i, 0)),
        out_shape=jax.ShapeDtypeStruct(a.shape, jnp.float32),
    )(a, b, bias.reshape(1, D))


def _gat_layer(x, src, dst, lp):
    h = x @ lp["W"]
    a_src_flat = lp["a_src"].reshape(D)
    a_dst_flat = lp["a_dst"].reshape(D)
    s_src = (h * a_src_flat).reshape(N, H, DH).sum(-1)
    s_dst = (h * a_dst_flat).reshape(N, H, DH).sum(-1)
    es = s_src[src] + s_dst[dst]
    es = jax.nn.leaky_relu(es, 0.2)
    alpha = _seg_softmax(es, dst, N)
    h3 = h.reshape(N, H, DH)
    out = jax.ops.segment_sum(
        (alpha[:, :, None] * h3[src]).reshape(E4, D), dst, num_segments=N
    )
    return jax.nn.elu(out)


def _gt_layer(x, src, dst, p):
    q = (x @ p["Wq"]).reshape(N, H, DH)
    k = (x @ p["Wk"]).reshape(N, H, DH)
    v = (x @ p["Wv"]).reshape(N, H, DH)
    sc = jnp.sum(q[dst] * k[src], -1) / np.sqrt(DH)
    alpha = _seg_softmax(sc, dst, N)
    agg = jax.ops.segment_sum(
        (alpha[:, :, None] * v[src]).reshape(E4, D), dst, num_segments=N
    ) @ p["Wo"]
    x1 = _ln(x + agg, p["ln1_s"], p["ln1_b"])
    ff = jax.nn.relu(x1 @ p["W1"] + p["b1"]) @ p["W2"] + p["b2"]
    return _ln(x1 + ff, p["ln2_s"], p["ln2_b"])


def _edge_mlps(x_ab, x_ag, ep, pairs):
    Wtop = ep["W"][:D]
    Wbot = ep["W"][D:]
    A = {"ab": x_ab @ Wtop, "ag": x_ag @ Wtop}
    B = {"ab": x_ab @ Wbot, "ag": x_ag @ Wbot}
    outs = []
    for s_name, d_name, s_idx, d_idx in pairs:
        outs.append(_relu_add_pallas(A[s_name][s_idx], B[d_name][d_idx], ep["b"]))
    return outs


@jax.jit
def kernel(x_ab, x_ag, pe_ab, pe_ag, params, edge_index_abag, edge_index_agab,
           edge_index_abab, edge_index_agag):
    src_all = jnp.concatenate([
        edge_index_abag[0], edge_index_agab[0] + N_AB,
        edge_index_abab[0], edge_index_agag[0] + N_AB])
    dst_all = jnp.concatenate([
        edge_index_abag[1] + N_AB, edge_index_agab[1],
        edge_index_abab[1], edge_index_agag[1] + N_AB])

    y_abag = y_agab = y_abab = y_agag = None
    for j in range(BLOCKS):
        x_ab = x_ab + pe_ab
        x_ag = x_ag + pe_ag
        x = jnp.concatenate([x_ab, x_ag], 0)
        for l in range(GAT_LAYERS):
            x = _gat_layer(x, src_all, dst_all, params["gat"][j][l])
        x_ab, x_ag = x[:N_AB], x[N_AB:]
        ep = params["int_edge"][j]
        e0, e1 = _edge_mlps(x_ab, x_ag, ep, [
            ("ab", "ag", edge_index_abag[0], edge_index_abag[1]),
            ("ag", "ab", edge_index_agab[0], edge_index_agab[1]),
        ])
        if j > 0:
            e0 = e0 + y_abag
            e1 = e1 + y_agab
        y_abag, y_agab = e0, e1
        x = jnp.concatenate([x_ab, x_ag], 0)
        x = _gt_layer(x, src_all, dst_all, params["gt"][j])
        x_ab, x_ag = x[:N_AB], x[N_AB:]
        ap = params["all_edge"][j]
        y_abag, y_agab, y_abab, y_agag = _edge_mlps(x_ab, x_ag, ap, [
            ("ab", "ag", edge_index_abag[0], edge_index_abag[1]),
            ("ag", "ab", edge_index_agab[0], edge_index_agab[1]),
            ("ab", "ab", edge_index_abab[0], edge_index_abab[1]),
            ("ag", "ag", edge_index_agag[0], edge_index_agag[1]),
        ])
    return (x_ab, x_ag, y_abag, y_agab, y_abab, y_agag)


# speedup vs baseline: 9.9138x; 9.9138x over previous
import functools

import jax
import jax.numpy as jnp
import numpy as np
from jax import lax
from jax.experimental import pallas as pl
from jax.experimental.pallas import tpu as pltpu
from jax.experimental.pallas import tpu_sc as plsc

N_AB = 5000
N_AG = 5000
N = N_AB + N_AG
E = 40000
E4 = 4 * E
D = 256
H = 8
DH = D // H
BLOCKS = 3
GAT_LAYERS = 2

B = 256
W = B + 8
NB = E4 // B
RMAXP = 81 * 128
NP = RMAXP
NEG = -1e30


_NW = 32
_CH = 128


def _sc_gather(table, idx2, dt):
    nc = idx2.shape[0]
    tpw = -(-nc // _NW)

    def body(table_hbm, idx_hbm, out_hbm, idx_v, rows_v, sem):
        wid = lax.axis_index("s") * 2 + lax.axis_index("c")

        def step(t, carry):
            cid = t * _NW + wid

            @pl.when(cid < nc)
            def _():
                pltpu.sync_copy(idx_hbm.at[cid], idx_v)
                pltpu.async_copy(table_hbm.at[idx_v], rows_v, sem).wait()
                pltpu.sync_copy(rows_v, out_hbm.at[pl.ds(cid * _CH, _CH)])

            return carry

        lax.fori_loop(0, tpw, step, 0)

    return pl.kernel(
        body,
        out_type=jax.ShapeDtypeStruct((nc * _CH, dt), jnp.float32),
        mesh=plsc.VectorSubcoreMesh(core_axis_name="c", subcore_axis_name="s"),
        scratch_types=[
            pltpu.VMEM((_CH,), jnp.int32),
            pltpu.VMEM((_CH, dt), jnp.float32),
            pltpu.SemaphoreType.DMA,
        ],
    )(table, idx2)


BN = 400
NDB = N // BN


def _full(shape):
    return pl.BlockSpec(shape, lambda *_: tuple(0 for _ in shape))


def _gat_dense(x, pe, Wm, a_dst_mat128):
    def body(x_ref, pe_ref, w_ref, adm_ref, g_ref, t_ref):
        xb = x_ref[...] + pe_ref[...]
        h = jnp.dot(xb, w_ref[...], preferred_element_type=jnp.float32)
        g_ref[...] = h
        t_ref[...] = jnp.dot(h, adm_ref[...],
                             preferred_element_type=jnp.float32)

    return pl.pallas_call(
        body,
        grid=(NDB,),
        in_specs=[
            pl.BlockSpec((BN, D), lambda i: (i, 0)),
            pl.BlockSpec((BN, D), lambda i: (i, 0)),
            _full((D, D)),
            _full((D, 128)),
        ],
        out_specs=[
            pl.BlockSpec((BN, D), lambda i: (i, 0)),
            pl.BlockSpec((BN, 128), lambda i: (i, 0)),
        ],
        out_shape=[
            jax.ShapeDtypeStruct((N, D), jnp.float32),
            jax.ShapeDtypeStruct((N, 128), jnp.float32),
        ],
    )(x, pe, Wm, a_dst_mat128)


def _qkv_dense(x, Wq, Wk, Wv):
    def body(x_ref, wq_ref, wk_ref, wv_ref, kv_ref, q_ref):
        xb = x_ref[...]
        kv_ref[:, 0:256] = jnp.dot(xb, wk_ref[...],
                                   preferred_element_type=jnp.float32)
        kv_ref[:, 256:512] = jnp.dot(xb, wv_ref[...],
                                     preferred_element_type=jnp.float32)
        q_ref[...] = jnp.dot(xb, wq_ref[...],
                             preferred_element_type=jnp.float32)

    return pl.pallas_call(
        body,
        grid=(NDB,),
        in_specs=[pl.BlockSpec((BN, D), lambda i: (i, 0)),
                  _full((D, D)), _full((D, D)), _full((D, D))],
        out_specs=[pl.BlockSpec((BN, 512), lambda i: (i, 0)),
                   pl.BlockSpec((BN, D), lambda i: (i, 0))],
        out_shape=[jax.ShapeDtypeStruct((N, 512), jnp.float32),
                   jax.ShapeDtypeStruct((N, D), jnp.float32)],
    )(x, Wq, Wk, Wv)


def _gt_post(x, agg, p):
    def body(x_ref, a_ref, wo_ref, w1_ref, w2_ref, v_ref, o_ref):
        ln1_s = v_ref[0:1, 0:256]
        ln1_b = v_ref[1:2, 0:256]
        ln2_s = v_ref[2:3, 0:256]
        ln2_b = v_ref[3:4, 0:256]
        b1 = v_ref[4:5, 0:512]
        b2 = v_ref[5:6, 0:256]
        xb = x_ref[...]
        a = xb + jnp.dot(a_ref[...], wo_ref[...],
                         preferred_element_type=jnp.float32)
        mu = jnp.mean(a, axis=1, keepdims=True)
        var = jnp.mean((a - mu) ** 2, axis=1, keepdims=True)
        x1 = (a - mu) / jnp.sqrt(var + 1e-5) * ln1_s + ln1_b
        f = jnp.maximum(jnp.dot(x1, w1_ref[...],
                                preferred_element_type=jnp.float32) + b1, 0.0)
        f = jnp.dot(f, w2_ref[...], preferred_element_type=jnp.float32) + b2
        a2 = x1 + f
        mu2 = jnp.mean(a2, axis=1, keepdims=True)
        var2 = jnp.mean((a2 - mu2) ** 2, axis=1, keepdims=True)
        o_ref[...] = (a2 - mu2) / jnp.sqrt(var2 + 1e-5) * ln2_s + ln2_b

    vecs = jnp.zeros((6, 512), jnp.float32)
    vecs = vecs.at[0, 0:256].set(p["ln1_s"])
    vecs = vecs.at[1, 0:256].set(p["ln1_b"])
    vecs = vecs.at[2, 0:256].set(p["ln2_s"])
    vecs = vecs.at[3, 0:256].set(p["ln2_b"])
    vecs = vecs.at[4, 0:512].set(p["b1"])
    vecs = vecs.at[5, 0:256].set(p["b2"])
    return pl.pallas_call(
        body,
        grid=(NDB,),
        in_specs=[pl.BlockSpec((BN, D), lambda i: (i, 0)),
                  pl.BlockSpec((BN, D), lambda i: (i, 0)),
                  _full((D, D)), _full((D, 2 * D)), _full((2 * D, D)),
                  _full((6, 512))],
        out_specs=pl.BlockSpec((BN, D), lambda i: (i, 0)),
        out_shape=jax.ShapeDtypeStruct((N, D), jnp.float32),
    )(x, agg, p["Wo"], p["W1"], p["W2"], vecs)


def _ab_dense(x, Wtop, Wbot):
    def body(x_ref, wt_ref, wb_ref, a_ref, b_ref):
        xb = x_ref[...]
        a_ref[...] = jnp.dot(xb, wt_ref[...],
                             preferred_element_type=jnp.float32)
        b_ref[...] = jnp.dot(xb, wb_ref[...],
                             preferred_element_type=jnp.float32)

    return pl.pallas_call(
        body,
        grid=(NDB,),
        in_specs=[pl.BlockSpec((BN, D), lambda i: (i, 0)),
                  _full((D, D)), _full((D, D))],
        out_specs=[pl.BlockSpec((BN, D), lambda i: (i, 0)),
                   pl.BlockSpec((BN, D), lambda i: (i, 0))],
        out_shape=[jax.ShapeDtypeStruct((N, D), jnp.float32),
                   jax.ShapeDtypeStruct((N, D), jnp.float32)],
    )(x, Wtop, Wbot)


def _relu_add(ga, gb, bias):
    ne = ga.shape[0]
    bn = 2000

    def body(a_ref, b_ref, v_ref, o_ref):
        o_ref[...] = jnp.maximum(a_ref[...] + b_ref[...] + v_ref[...], 0.0)

    return pl.pallas_call(
        body,
        grid=(ne // bn,),
        in_specs=[pl.BlockSpec((bn, D), lambda i: (i, 0)),
                  pl.BlockSpec((bn, D), lambda i: (i, 0)),
                  _full((1, D))],
        out_specs=pl.BlockSpec((bn, D), lambda i: (i, 0)),
        out_shape=jax.ShapeDtypeStruct((ne, D), jnp.float32),
    )(ga, gb, bias.reshape(1, D))


def _onehot_t(loc2d):
    iw = lax.broadcasted_iota(jnp.int32, (W, B), 0)
    mb = iw == loc2d
    return mb.astype(jnp.float32), mb


def _masked_seg_max(mb, es):
    cols = []
    for h in range(H):
        t = jnp.where(mb, es[:, h].reshape(1, B), NEG)
        cols.append(jnp.max(t, axis=1)[:, None])
    return jnp.concatenate(cols, axis=1)


def _expand_mat():
    r = lax.broadcasted_iota(jnp.int32, (H, D), 0)
    c = lax.broadcasted_iota(jnp.int32, (H, D), 1)
    return (c // DH == r).astype(jnp.float32)


def _gat_pass1(ge, sdr, asm, loc3, r0a):
    def body(s_ref, g_ref, sdr_ref, asm_ref, loc_ref, m_ref):
        i = pl.program_id(0)

        @pl.when(i == 0)
        def _():
            m_ref[...] = jnp.full((RMAXP, H), NEG, jnp.float32)

        r0 = pl.multiple_of(s_ref[i], 8)
        mt, mb = _onehot_t(loc_ref[0])
        sd_sl = sdr_ref[pl.ds(r0, W), 0:8]
        sd_e = lax.dot_general(mt, sd_sl, (((0,), (0,)), ((), ())),
                               preferred_element_type=jnp.float32)
        ss = jnp.dot(g_ref[...], asm_ref[...],
                     preferred_element_type=jnp.float32)
        es = ss + sd_e
        es = jnp.where(es >= 0, es, 0.2 * es)
        bm = _masked_seg_max(mb, es)
        m_ref[pl.ds(r0, W), :] = jnp.maximum(m_ref[pl.ds(r0, W), :], bm)

    grid_spec = pltpu.PrefetchScalarGridSpec(
        num_scalar_prefetch=1,
        grid=(NB,),
        in_specs=[
            pl.BlockSpec((B, D), lambda i, s: (i, 0)),
            _full((RMAXP, 128)),
            _full((D, H)),
            pl.BlockSpec((1, 1, B), lambda i, s: (i, 0, 0)),
        ],
        out_specs=_full((RMAXP, H)),
    )
    return pl.pallas_call(
        body, grid_spec=grid_spec,
        out_shape=jax.ShapeDtypeStruct((RMAXP, H), jnp.float32),
    )(r0a, ge, sdr, asm, loc3)


def _gat_pass2(ge, sdr, asm, m, loc3, r0a):
    def body(s_ref, g_ref, sdr_ref, asm_ref, m_ref, loc_ref, den_ref, num_ref):
        i = pl.program_id(0)

        @pl.when(i == 0)
        def _():
            den_ref[...] = jnp.zeros((RMAXP, H), jnp.float32)
            num_ref[...] = jnp.zeros((RMAXP, D), jnp.float32)

        r0 = pl.multiple_of(s_ref[i], 8)
        mt, mb = _onehot_t(loc_ref[0])
        g = g_ref[...]
        sd_sl = sdr_ref[pl.ds(r0, W), 0:8]
        sd_e = lax.dot_general(mt, sd_sl, (((0,), (0,)), ((), ())),
                               preferred_element_type=jnp.float32)
        es = jnp.dot(g, asm_ref[...],
                     preferred_element_type=jnp.float32) + sd_e
        es = jnp.where(es >= 0, es, 0.2 * es)
        m_sl = m_ref[pl.ds(r0, W), :]
        m_e = lax.dot_general(mt, m_sl, (((0,), (0,)), ((), ())),
                              preferred_element_type=jnp.float32)
        ex = jnp.exp(es - m_e)
        den_ref[pl.ds(r0, W), :] += lax.dot_general(
            mt, ex, (((1,), (0,)), ((), ())),
            preferred_element_type=jnp.float32)
        ex_wide = lax.dot_general(ex, _expand_mat(), (((1,), (0,)), ((), ())),
                                  preferred_element_type=jnp.float32)
        vals = g * ex_wide
        num_ref[pl.ds(r0, W), :] += lax.dot_general(
            mt, vals, (((1,), (0,)), ((), ())),
            preferred_element_type=jnp.float32)

    grid_spec = pltpu.PrefetchScalarGridSpec(
        num_scalar_prefetch=1,
        grid=(NB,),
        in_specs=[
            pl.BlockSpec((B, D), lambda i, s: (i, 0)),
            _full((RMAXP, 128)),
            _full((D, H)),
            _full((RMAXP, H)),
            pl.BlockSpec((1, 1, B), lambda i, s: (i, 0, 0)),
        ],
        out_specs=[_full((RMAXP, H)), _full((RMAXP, D))],
    )
    return pl.pallas_call(
        body, grid_spec=grid_spec,
        out_shape=[jax.ShapeDtypeStruct((RMAXP, H), jnp.float32),
                   jax.ShapeDtypeStruct((RMAXP, D), jnp.float32)],
    )(r0a, ge, sdr, asm, m, loc3)


def _gt_pass1(gkv, qr, loc3, r0a):
    inv = float(1.0 / np.sqrt(DH))

    def body(s_ref, k_ref, qr_ref, loc_ref, m_ref, sc_ref):
        i = pl.program_id(0)

        @pl.when(i == 0)
        def _():
            m_ref[...] = jnp.full((RMAXP, H), NEG, jnp.float32)

        r0 = pl.multiple_of(s_ref[i], 8)
        mt, mb = _onehot_t(loc_ref[0])
        q_sl = qr_ref[pl.ds(r0, W), :]
        q_e = lax.dot_general(mt, q_sl, (((0,), (0,)), ((), ())),
                              preferred_element_type=jnp.float32)
        qk = q_e * k_ref[...]
        sc = lax.dot_general(qk, _expand_mat(), (((1,), (1,)), ((), ())),
                             preferred_element_type=jnp.float32) * inv
        sc_ref[...] = sc
        bm = _masked_seg_max(mb, sc)
        m_ref[pl.ds(r0, W), :] = jnp.maximum(m_ref[pl.ds(r0, W), :], bm)

    grid_spec = pltpu.PrefetchScalarGridSpec(
        num_scalar_prefetch=1,
        grid=(NB,),
        in_specs=[
            pl.BlockSpec((B, 256), lambda i, s: (i, 0)),
            _full((RMAXP, D)),
            pl.BlockSpec((1, 1, B), lambda i, s: (i, 0, 0)),
        ],
        out_specs=[_full((RMAXP, H)),
                   pl.BlockSpec((B, H), lambda i, s: (i, 0))],
    )
    return pl.pallas_call(
        body, grid_spec=grid_spec,
        out_shape=[jax.ShapeDtypeStruct((RMAXP, H), jnp.float32),
                   jax.ShapeDtypeStruct((E4, H), jnp.float32)],
    )(r0a, gkv, qr, loc3)


def _gt_pass2(gkv, sc, m, loc3, r0a):
    def body(s_ref, v_ref, sc_ref, m_ref, loc_ref, den_ref, num_ref):
        i = pl.program_id(0)

        @pl.when(i == 0)
        def _():
            den_ref[...] = jnp.zeros((RMAXP, H), jnp.float32)
            num_ref[...] = jnp.zeros((RMAXP, D), jnp.float32)

        r0 = pl.multiple_of(s_ref[i], 8)
        mt, mb = _onehot_t(loc_ref[0])
        m_sl = m_ref[pl.ds(r0, W), :]
        m_e = lax.dot_general(mt, m_sl, (((0,), (0,)), ((), ())),
                              preferred_element_type=jnp.float32)
        ex = jnp.exp(sc_ref[...] - m_e)
        den_ref[pl.ds(r0, W), :] += lax.dot_general(
            mt, ex, (((1,), (0,)), ((), ())),
            preferred_element_type=jnp.float32)
        ex_wide = lax.dot_general(ex, _expand_mat(), (((1,), (0,)), ((), ())),
                                  preferred_element_type=jnp.float32)
        vals = v_ref[...] * ex_wide
        num_ref[pl.ds(r0, W), :] += lax.dot_general(
            mt, vals, (((1,), (0,)), ((), ())),
            preferred_element_type=jnp.float32)

    grid_spec = pltpu.PrefetchScalarGridSpec(
        num_scalar_prefetch=1,
        grid=(NB,),
        in_specs=[
            pl.BlockSpec((B, 256), lambda i, s: (i, 1)),
            pl.BlockSpec((B, H), lambda i, s: (i, 0)),
            _full((RMAXP, H)),
            pl.BlockSpec((1, 1, B), lambda i, s: (i, 0, 0)),
        ],
        out_specs=[_full((RMAXP, H)), _full((RMAXP, D))],
    )
    return pl.pallas_call(
        body, grid_spec=grid_spec,
        out_shape=[jax.ShapeDtypeStruct((RMAXP, H), jnp.float32),
                   jax.ShapeDtypeStruct((RMAXP, D), jnp.float32)],
    )(r0a, gkv, sc, m, loc3)


def _finalize(num, den, act):
    rb = 128

    def body(n_ref, d_ref, o_ref):
        d_wide = lax.dot_general(d_ref[...], _expand_mat(),
                                 (((1,), (0,)), ((), ())),
                                 preferred_element_type=jnp.float32)
        v = n_ref[...] / (d_wide + 1e-9)
        if act == "elu":
            v = jnp.where(v > 0, v, jnp.exp(jnp.minimum(v, 0.0)) - 1.0)
        o_ref[...] = v

    return pl.pallas_call(
        body,
        grid=(RMAXP // rb,),
        in_specs=[pl.BlockSpec((rb, D), lambda i: (i, 0)),
                  pl.BlockSpec((rb, H), lambda i: (i, 0))],
        out_specs=pl.BlockSpec((rb, D), lambda i: (i, 0)),
        out_shape=jax.ShapeDtypeStruct((RMAXP, D), jnp.float32),
    )(num, den)


def _prep_indices(src_all, dst_all):
    perm = jnp.argsort(dst_all)
    dst_s = dst_all[perm]
    src_s = src_all[perm]
    f0 = jnp.concatenate([
        jnp.ones((1,), jnp.int32),
        (dst_s[1:] != dst_s[:-1]).astype(jnp.int32)])
    rank = jnp.cumsum(f0) - 1
    nrank = rank[-1] + 1
    ar = jnp.arange(RMAXP, dtype=jnp.int32)
    node_of_rank = jnp.zeros((RMAXP,), jnp.int32).at[rank].set(dst_s)
    node_of_rank = jnp.where(ar < nrank, node_of_rank, 0)
    has_edge = jnp.zeros((N,), jnp.bool_).at[dst_s].set(True)
    rank_of_node = jnp.zeros((N,), jnp.int32).at[dst_s].set(rank)
    rank_of_node = jnp.where(has_edge, rank_of_node, RMAXP - 1)
    rank_of_node_p = jnp.concatenate(
        [rank_of_node, jnp.zeros((NP - N,), jnp.int32)])
    r0a = (rank[::B] // 8) * 8
    loc = rank - jnp.repeat(r0a, B)
    return {
        "src_s2": src_s.reshape(E4 // _CH, _CH),
        "nor2": node_of_rank.reshape(RMAXP // _CH, _CH),
        "ron2": rank_of_node_p.reshape(NP // _CH, _CH),
        "r0a": r0a.astype(jnp.int32),
        "loc3": loc.reshape(NB, 1, B).astype(jnp.int32),
    }


def _gat_layer(x, pe, lp, ix):
    asm = _head_mat(lp["a_src"], H)
    adm = _head_mat(lp["a_dst"], 128)
    th, tsd = _gat_dense(x, pe, lp["W"], adm)
    sdr = _sc_gather(tsd, ix["nor2"], 128)
    ge = _sc_gather(th, ix["src_s2"], 256)
    m = _gat_pass1(ge, sdr, asm, ix["loc3"], ix["r0a"])
    den, num = _gat_pass2(ge, sdr, asm, m, ix["loc3"], ix["r0a"])
    fin = _finalize(num, den, "elu")
    return _sc_gather(fin, ix["ron2"], 256)[:N]


def _gt_layer(x, p, ix):
    kv, q = _qkv_dense(x, p["Wq"], p["Wk"], p["Wv"])
    qr = _sc_gather(q, ix["nor2"], 256)
    gkv = _sc_gather(kv, ix["src_s2"], 512)
    m, sc = _gt_pass1(gkv, qr, ix["loc3"], ix["r0a"])
    den, num = _gt_pass2(gkv, sc, m, ix["loc3"], ix["r0a"])
    fin = _finalize(num, den, "none")
    agg = _sc_gather(fin, ix["ron2"], 256)[:N]
    return _gt_post(x, agg, p)


def _head_mat(a, width):
    c = jnp.arange(D)
    hsel = (c // DH)[:, None] == jnp.arange(width)[None, :]
    vals = a.reshape(D)[:, None]
    return jnp.where(hsel, vals, 0.0).astype(jnp.float32)


@jax.jit
def kernel(x_ab, x_ag, pe_ab, pe_ag, params, edge_index_abag, edge_index_agab,
           edge_index_abab, edge_index_agag):
    src_all = jnp.concatenate([
        edge_index_abag[0], edge_index_agab[0] + N_AB,
        edge_index_abab[0], edge_index_agag[0] + N_AB]).astype(jnp.int32)
    dst_all = jnp.concatenate([
        edge_index_abag[1] + N_AB, edge_index_agab[1],
        edge_index_abab[1], edge_index_agag[1] + N_AB]).astype(jnp.int32)
    ix = _prep_indices(src_all, dst_all)

    pe = jnp.concatenate([pe_ab, pe_ag], 0)
    zeros_pe = jnp.zeros_like(pe)
    x = jnp.concatenate([x_ab, x_ag], 0)

    for j in range(BLOCKS):
        for l in range(GAT_LAYERS):
            x = _gat_layer(x, pe if l == 0 else zeros_pe,
                           params["gat"][j][l], ix)
        x = _gt_layer(x, params["gt"][j], ix)

    ap = params["all_edge"][BLOCKS - 1]
    a_t, b_t = _ab_dense(x, ap["W"][:D], ap["W"][D:])
    s_cat = jnp.concatenate([
        edge_index_abag[0], edge_index_agab[0] + N_AB,
        edge_index_abab[0], edge_index_agag[0] + N_AB]).astype(jnp.int32)
    d_cat = jnp.concatenate([
        edge_index_abag[1] + N_AB, edge_index_agab[1],
        edge_index_abab[1], edge_index_agag[1] + N_AB]).astype(jnp.int32)
    ga = _sc_gather(a_t, s_cat.reshape(E4 // _CH, _CH), 256)
    gb = _sc_gather(b_t, d_cat.reshape(E4 // _CH, _CH), 256)
    ecat = _relu_add(ga, gb, ap["b"])
    y_abag = ecat[0:E]
    y_agab = ecat[E:2 * E]
    y_abab = ecat[2 * E:3 * E]
    y_agag = ecat[3 * E:4 * E]
    return (x[:N_AB], x[N_AB:], y_abag, y_agab, y_abab, y_agag)


# speedup vs baseline: 11.3648x; 1.1464x over previous
import functools

import jax
import jax.numpy as jnp
import numpy as np
from jax import lax
from jax.experimental import pallas as pl
from jax.experimental.pallas import tpu as pltpu
from jax.experimental.pallas import tpu_sc as plsc

N_AB = 5000
N_AG = 5000
N = N_AB + N_AG
E = 40000
E4 = 4 * E
D = 256
H = 8
DH = D // H
BLOCKS = 3
GAT_LAYERS = 2

B = 256
W = B + 8
NB = E4 // B
UB = 5
NB2 = NB // UB
RMAXP = 81 * 128
NP = RMAXP
NEG = -1e30


_NW = 32
_CH = 128


def _sc_gather(table, idx2, dt):
    nc = idx2.shape[0]
    tpw = -(-nc // _NW)

    def body(table_hbm, idx_hbm, out_hbm, idx_v, rows_v, sem):
        wid = lax.axis_index("s") * 2 + lax.axis_index("c")

        def step(t, carry):
            cid = t * _NW + wid

            @pl.when(cid < nc)
            def _():
                pltpu.sync_copy(idx_hbm.at[cid], idx_v)
                pltpu.async_copy(table_hbm.at[idx_v], rows_v, sem).wait()
                pltpu.sync_copy(rows_v, out_hbm.at[pl.ds(cid * _CH, _CH)])

            return carry

        lax.fori_loop(0, tpw, step, 0)

    return pl.kernel(
        body,
        out_type=jax.ShapeDtypeStruct((nc * _CH, dt), jnp.float32),
        mesh=plsc.VectorSubcoreMesh(core_axis_name="c", subcore_axis_name="s"),
        scratch_types=[
            pltpu.VMEM((_CH,), jnp.int32),
            pltpu.VMEM((_CH, dt), jnp.float32),
            pltpu.SemaphoreType.DMA,
        ],
    )(table, idx2)


BN = 400
NDB = N // BN


def _full(shape):
    return pl.BlockSpec(shape, lambda *_: tuple(0 for _ in shape))


def _gat_dense(x, pe, Wm, a_dst_mat128):
    def body(x_ref, pe_ref, w_ref, adm_ref, g_ref, t_ref):
        xb = x_ref[...] + pe_ref[...]
        h = jnp.dot(xb, w_ref[...], preferred_element_type=jnp.float32)
        g_ref[...] = h
        t_ref[...] = jnp.dot(h, adm_ref[...],
                             preferred_element_type=jnp.float32)

    return pl.pallas_call(
        body,
        grid=(NDB,),
        in_specs=[
            pl.BlockSpec((BN, D), lambda i: (i, 0)),
            pl.BlockSpec((BN, D), lambda i: (i, 0)),
            _full((D, D)),
            _full((D, 128)),
        ],
        out_specs=[
            pl.BlockSpec((BN, D), lambda i: (i, 0)),
            pl.BlockSpec((BN, 128), lambda i: (i, 0)),
        ],
        out_shape=[
            jax.ShapeDtypeStruct((N, D), jnp.float32),
            jax.ShapeDtypeStruct((N, 128), jnp.float32),
        ],
    )(x, pe, Wm, a_dst_mat128)


def _qkv_dense(x, Wq, Wk, Wv):
    def body(x_ref, wq_ref, wk_ref, wv_ref, kv_ref, q_ref):
        xb = x_ref[...]
        kv_ref[:, 0:256] = jnp.dot(xb, wk_ref[...],
                                   preferred_element_type=jnp.float32)
        kv_ref[:, 256:512] = jnp.dot(xb, wv_ref[...],
                                     preferred_element_type=jnp.float32)
        q_ref[...] = jnp.dot(xb, wq_ref[...],
                             preferred_element_type=jnp.float32)

    return pl.pallas_call(
        body,
        grid=(NDB,),
        in_specs=[pl.BlockSpec((BN, D), lambda i: (i, 0)),
                  _full((D, D)), _full((D, D)), _full((D, D))],
        out_specs=[pl.BlockSpec((BN, 512), lambda i: (i, 0)),
                   pl.BlockSpec((BN, D), lambda i: (i, 0))],
        out_shape=[jax.ShapeDtypeStruct((N, 512), jnp.float32),
                   jax.ShapeDtypeStruct((N, D), jnp.float32)],
    )(x, Wq, Wk, Wv)


def _gt_post(x, agg, p):
    def body(x_ref, a_ref, wo_ref, w1_ref, w2_ref, v_ref, o_ref):
        ln1_s = v_ref[0:1, 0:256]
        ln1_b = v_ref[1:2, 0:256]
        ln2_s = v_ref[2:3, 0:256]
        ln2_b = v_ref[3:4, 0:256]
        b1 = v_ref[4:5, 0:512]
        b2 = v_ref[5:6, 0:256]
        xb = x_ref[...]
        a = xb + jnp.dot(a_ref[...], wo_ref[...],
                         preferred_element_type=jnp.float32)
        mu = jnp.mean(a, axis=1, keepdims=True)
        var = jnp.mean((a - mu) ** 2, axis=1, keepdims=True)
        x1 = (a - mu) / jnp.sqrt(var + 1e-5) * ln1_s + ln1_b
        f = jnp.maximum(jnp.dot(x1, w1_ref[...],
                                preferred_element_type=jnp.float32) + b1, 0.0)
        f = jnp.dot(f, w2_ref[...], preferred_element_type=jnp.float32) + b2
        a2 = x1 + f
        mu2 = jnp.mean(a2, axis=1, keepdims=True)
        var2 = jnp.mean((a2 - mu2) ** 2, axis=1, keepdims=True)
        o_ref[...] = (a2 - mu2) / jnp.sqrt(var2 + 1e-5) * ln2_s + ln2_b

    vecs = jnp.zeros((6, 512), jnp.float32)
    vecs = vecs.at[0, 0:256].set(p["ln1_s"])
    vecs = vecs.at[1, 0:256].set(p["ln1_b"])
    vecs = vecs.at[2, 0:256].set(p["ln2_s"])
    vecs = vecs.at[3, 0:256].set(p["ln2_b"])
    vecs = vecs.at[4, 0:512].set(p["b1"])
    vecs = vecs.at[5, 0:256].set(p["b2"])
    return pl.pallas_call(
        body,
        grid=(NDB,),
        in_specs=[pl.BlockSpec((BN, D), lambda i: (i, 0)),
                  pl.BlockSpec((BN, D), lambda i: (i, 0)),
                  _full((D, D)), _full((D, 2 * D)), _full((2 * D, D)),
                  _full((6, 512))],
        out_specs=pl.BlockSpec((BN, D), lambda i: (i, 0)),
        out_shape=jax.ShapeDtypeStruct((N, D), jnp.float32),
    )(x, agg, p["Wo"], p["W1"], p["W2"], vecs)


def _ab_dense(x, Wtop, Wbot):
    def body(x_ref, wt_ref, wb_ref, a_ref, b_ref):
        xb = x_ref[...]
        a_ref[...] = jnp.dot(xb, wt_ref[...],
                             preferred_element_type=jnp.float32)
        b_ref[...] = jnp.dot(xb, wb_ref[...],
                             preferred_element_type=jnp.float32)

    return pl.pallas_call(
        body,
        grid=(NDB,),
        in_specs=[pl.BlockSpec((BN, D), lambda i: (i, 0)),
                  _full((D, D)), _full((D, D))],
        out_specs=[pl.BlockSpec((BN, D), lambda i: (i, 0)),
                   pl.BlockSpec((BN, D), lambda i: (i, 0))],
        out_shape=[jax.ShapeDtypeStruct((N, D), jnp.float32),
                   jax.ShapeDtypeStruct((N, D), jnp.float32)],
    )(x, Wtop, Wbot)


def _relu_add(ga, gb, bias):
    ne = ga.shape[0]
    bn = 2000

    def body(a_ref, b_ref, v_ref, o_ref):
        o_ref[...] = jnp.maximum(a_ref[...] + b_ref[...] + v_ref[...], 0.0)

    return pl.pallas_call(
        body,
        grid=(ne // bn,),
        in_specs=[pl.BlockSpec((bn, D), lambda i: (i, 0)),
                  pl.BlockSpec((bn, D), lambda i: (i, 0)),
                  _full((1, D))],
        out_specs=pl.BlockSpec((bn, D), lambda i: (i, 0)),
        out_shape=jax.ShapeDtypeStruct((ne, D), jnp.float32),
    )(ga, gb, bias.reshape(1, D))


def _onehot_t(loc2d):
    iw = lax.broadcasted_iota(jnp.int32, (W, B), 0)
    mb = iw == loc2d
    return mb.astype(jnp.float32), mb


def _masked_seg_max(mb, es):
    cols = []
    for h in range(H):
        t = jnp.where(mb, es[:, h].reshape(1, B), NEG)
        cols.append(jnp.max(t, axis=1)[:, None])
    return jnp.concatenate(cols, axis=1)


def _expand_mat():
    r = lax.broadcasted_iota(jnp.int32, (H, D), 0)
    c = lax.broadcasted_iota(jnp.int32, (H, D), 1)
    return (c // DH == r).astype(jnp.float32)


def _gat_pass1(ge, sdr, asm, loc3, r0a):
    def body(s_ref, g_ref, sdr_ref, asm_ref, loc_ref, m_ref, es_ref):
        i = pl.program_id(0)

        @pl.when(i == 0)
        def _():
            m_ref[...] = jnp.full((RMAXP, H), NEG, jnp.float32)

        for u in range(UB):
            r0 = pl.multiple_of(s_ref[i * UB + u], 8)
            mt, mb = _onehot_t(loc_ref[0][:, u * B:(u + 1) * B])
            sd_sl = sdr_ref[pl.ds(r0, W), 0:8]
            sd_e = lax.dot_general(mt, sd_sl, (((0,), (0,)), ((), ())),
                                   preferred_element_type=jnp.float32)
            ss = jnp.dot(g_ref[u * B:(u + 1) * B, :], asm_ref[...],
                         preferred_element_type=jnp.float32)
            es = ss + sd_e
            es = jnp.where(es >= 0, es, 0.2 * es)
            es_ref[u * B:(u + 1) * B, :] = es
            bm = _masked_seg_max(mb, es)
            m_ref[pl.ds(r0, W), :] = jnp.maximum(m_ref[pl.ds(r0, W), :], bm)

    grid_spec = pltpu.PrefetchScalarGridSpec(
        num_scalar_prefetch=1,
        grid=(NB2,),
        in_specs=[
            pl.BlockSpec((UB * B, D), lambda i, s: (i, 0)),
            _full((RMAXP, 128)),
            _full((D, H)),
            pl.BlockSpec((1, 1, UB * B), lambda i, s: (i, 0, 0)),
        ],
        out_specs=[_full((RMAXP, H)),
                   pl.BlockSpec((UB * B, H), lambda i, s: (i, 0))],
    )
    return pl.pallas_call(
        body, grid_spec=grid_spec,
        out_shape=[jax.ShapeDtypeStruct((RMAXP, H), jnp.float32),
                   jax.ShapeDtypeStruct((E4, H), jnp.float32)],
    )(r0a, ge, sdr, asm, loc3)


def _gat_pass2(ge, es_all, m, loc3, r0a):
    def body(s_ref, g_ref, es_ref, m_ref, loc_ref, den_ref, num_ref):
        i = pl.program_id(0)

        @pl.when(i == 0)
        def _():
            den_ref[...] = jnp.zeros((RMAXP, H), jnp.float32)
            num_ref[...] = jnp.zeros((RMAXP, D), jnp.float32)

        for u in range(UB):
            r0 = pl.multiple_of(s_ref[i * UB + u], 8)
            mt, mb = _onehot_t(loc_ref[0][:, u * B:(u + 1) * B])
            g = g_ref[u * B:(u + 1) * B, :]
            es = es_ref[u * B:(u + 1) * B, :]
            m_sl = m_ref[pl.ds(r0, W), :]
            m_e = lax.dot_general(mt, m_sl, (((0,), (0,)), ((), ())),
                                  preferred_element_type=jnp.float32)
            ex = jnp.exp(es - m_e)
            den_ref[pl.ds(r0, W), :] += lax.dot_general(
                mt, ex, (((1,), (0,)), ((), ())),
                preferred_element_type=jnp.float32)
            ex_wide = lax.dot_general(ex, _expand_mat(),
                                      (((1,), (0,)), ((), ())),
                                      preferred_element_type=jnp.float32)
            vals = g * ex_wide
            num_ref[pl.ds(r0, W), :] += lax.dot_general(
                mt, vals, (((1,), (0,)), ((), ())),
                preferred_element_type=jnp.float32)

    grid_spec = pltpu.PrefetchScalarGridSpec(
        num_scalar_prefetch=1,
        grid=(NB2,),
        in_specs=[
            pl.BlockSpec((UB * B, D), lambda i, s: (i, 0)),
            pl.BlockSpec((UB * B, H), lambda i, s: (i, 0)),
            _full((RMAXP, H)),
            pl.BlockSpec((1, 1, UB * B), lambda i, s: (i, 0, 0)),
        ],
        out_specs=[_full((RMAXP, H)), _full((RMAXP, D))],
    )
    return pl.pallas_call(
        body, grid_spec=grid_spec,
        out_shape=[jax.ShapeDtypeStruct((RMAXP, H), jnp.float32),
                   jax.ShapeDtypeStruct((RMAXP, D), jnp.float32)],
    )(r0a, ge, es_all, m, loc3)


def _gt_pass1(gkv, qr, loc3, r0a):
    inv = float(1.0 / np.sqrt(DH))

    def body(s_ref, k_ref, qr_ref, loc_ref, m_ref, sc_ref):
        i = pl.program_id(0)

        @pl.when(i == 0)
        def _():
            m_ref[...] = jnp.full((RMAXP, H), NEG, jnp.float32)

        for u in range(UB):
            r0 = pl.multiple_of(s_ref[i * UB + u], 8)
            mt, mb = _onehot_t(loc_ref[0][:, u * B:(u + 1) * B])
            q_sl = qr_ref[pl.ds(r0, W), :]
            q_e = lax.dot_general(mt, q_sl, (((0,), (0,)), ((), ())),
                                  preferred_element_type=jnp.float32)
            qk = q_e * k_ref[u * B:(u + 1) * B, :]
            sc = lax.dot_general(qk, _expand_mat(), (((1,), (1,)), ((), ())),
                                 preferred_element_type=jnp.float32) * inv
            sc_ref[u * B:(u + 1) * B, :] = sc
            bm = _masked_seg_max(mb, sc)
            m_ref[pl.ds(r0, W), :] = jnp.maximum(m_ref[pl.ds(r0, W), :], bm)

    grid_spec = pltpu.PrefetchScalarGridSpec(
        num_scalar_prefetch=1,
        grid=(NB2,),
        in_specs=[
            pl.BlockSpec((UB * B, 256), lambda i, s: (i, 0)),
            _full((RMAXP, D)),
            pl.BlockSpec((1, 1, UB * B), lambda i, s: (i, 0, 0)),
        ],
        out_specs=[_full((RMAXP, H)),
                   pl.BlockSpec((UB * B, H), lambda i, s: (i, 0))],
    )
    return pl.pallas_call(
        body, grid_spec=grid_spec,
        out_shape=[jax.ShapeDtypeStruct((RMAXP, H), jnp.float32),
                   jax.ShapeDtypeStruct((E4, H), jnp.float32)],
    )(r0a, gkv, qr, loc3)


def _gt_pass2(gkv, sc, m, loc3, r0a):
    def body(s_ref, v_ref, sc_ref, m_ref, loc_ref, den_ref, num_ref):
        i = pl.program_id(0)

        @pl.when(i == 0)
        def _():
            den_ref[...] = jnp.zeros((RMAXP, H), jnp.float32)
            num_ref[...] = jnp.zeros((RMAXP, D), jnp.float32)

        for u in range(UB):
            r0 = pl.multiple_of(s_ref[i * UB + u], 8)
            mt, mb = _onehot_t(loc_ref[0][:, u * B:(u + 1) * B])
            m_sl = m_ref[pl.ds(r0, W), :]
            m_e = lax.dot_general(mt, m_sl, (((0,), (0,)), ((), ())),
                                  preferred_element_type=jnp.float32)
            ex = jnp.exp(sc_ref[u * B:(u + 1) * B, :] - m_e)
            den_ref[pl.ds(r0, W), :] += lax.dot_general(
                mt, ex, (((1,), (0,)), ((), ())),
                preferred_element_type=jnp.float32)
            ex_wide = lax.dot_general(ex, _expand_mat(),
                                      (((1,), (0,)), ((), ())),
                                      preferred_element_type=jnp.float32)
            vals = v_ref[u * B:(u + 1) * B, :] * ex_wide
            num_ref[pl.ds(r0, W), :] += lax.dot_general(
                mt, vals, (((1,), (0,)), ((), ())),
                preferred_element_type=jnp.float32)

    grid_spec = pltpu.PrefetchScalarGridSpec(
        num_scalar_prefetch=1,
        grid=(NB2,),
        in_specs=[
            pl.BlockSpec((UB * B, 256), lambda i, s: (i, 1)),
            pl.BlockSpec((UB * B, H), lambda i, s: (i, 0)),
            _full((RMAXP, H)),
            pl.BlockSpec((1, 1, UB * B), lambda i, s: (i, 0, 0)),
        ],
        out_specs=[_full((RMAXP, H)), _full((RMAXP, D))],
    )
    return pl.pallas_call(
        body, grid_spec=grid_spec,
        out_shape=[jax.ShapeDtypeStruct((RMAXP, H), jnp.float32),
                   jax.ShapeDtypeStruct((RMAXP, D), jnp.float32)],
    )(r0a, gkv, sc, m, loc3)


def _finalize(num, den, act):
    rb = 128

    def body(n_ref, d_ref, o_ref):
        d_wide = lax.dot_general(d_ref[...], _expand_mat(),
                                 (((1,), (0,)), ((), ())),
                                 preferred_element_type=jnp.float32)
        v = n_ref[...] / (d_wide + 1e-9)
        if act == "elu":
            v = jnp.where(v > 0, v, jnp.exp(jnp.minimum(v, 0.0)) - 1.0)
        o_ref[...] = v

    return pl.pallas_call(
        body,
        grid=(RMAXP // rb,),
        in_specs=[pl.BlockSpec((rb, D), lambda i: (i, 0)),
                  pl.BlockSpec((rb, H), lambda i: (i, 0))],
        out_specs=pl.BlockSpec((rb, D), lambda i: (i, 0)),
        out_shape=jax.ShapeDtypeStruct((RMAXP, D), jnp.float32),
    )(num, den)


def _prep_indices(src_all, dst_all):
    perm = jnp.argsort(dst_all)
    dst_s = dst_all[perm]
    src_s = src_all[perm]
    f0 = jnp.concatenate([
        jnp.ones((1,), jnp.int32),
        (dst_s[1:] != dst_s[:-1]).astype(jnp.int32)])
    rank = jnp.cumsum(f0) - 1
    nrank = rank[-1] + 1
    ar = jnp.arange(RMAXP, dtype=jnp.int32)
    node_of_rank = jnp.zeros((RMAXP,), jnp.int32).at[rank].set(dst_s)
    node_of_rank = jnp.where(ar < nrank, node_of_rank, 0)
    has_edge = jnp.zeros((N,), jnp.bool_).at[dst_s].set(True)
    rank_of_node = jnp.zeros((N,), jnp.int32).at[dst_s].set(rank)
    rank_of_node = jnp.where(has_edge, rank_of_node, RMAXP - 1)
    rank_of_node_p = jnp.concatenate(
        [rank_of_node, jnp.zeros((NP - N,), jnp.int32)])
    r0a = (rank[::B] // 8) * 8
    loc = rank - jnp.repeat(r0a, B)
    return {
        "src_s2": src_s.reshape(E4 // _CH, _CH),
        "nor2": node_of_rank.reshape(RMAXP // _CH, _CH),
        "ron2": rank_of_node_p.reshape(NP // _CH, _CH),
        "r0a": r0a.astype(jnp.int32),
        "loc3": loc.reshape(NB2, 1, UB * B).astype(jnp.int32),
    }


def _gat_layer(x, pe, lp, ix):
    asm = _head_mat(lp["a_src"], H)
    adm = _head_mat(lp["a_dst"], 128)
    th, tsd = _gat_dense(x, pe, lp["W"], adm)
    sdr = _sc_gather(tsd, ix["nor2"], 128)
    ge = _sc_gather(th, ix["src_s2"], 256)
    m, es = _gat_pass1(ge, sdr, asm, ix["loc3"], ix["r0a"])
    den, num = _gat_pass2(ge, es, m, ix["loc3"], ix["r0a"])
    fin = _finalize(num, den, "elu")
    return _sc_gather(fin, ix["ron2"], 256)[:N]


def _gt_layer(x, p, ix):
    kv, q = _qkv_dense(x, p["Wq"], p["Wk"], p["Wv"])
    qr = _sc_gather(q, ix["nor2"], 256)
    gkv = _sc_gather(kv, ix["src_s2"], 512)
    m, sc = _gt_pass1(gkv, qr, ix["loc3"], ix["r0a"])
    den, num = _gt_pass2(gkv, sc, m, ix["loc3"], ix["r0a"])
    fin = _finalize(num, den, "none")
    agg = _sc_gather(fin, ix["ron2"], 256)[:N]
    return _gt_post(x, agg, p)


def _head_mat(a, width):
    c = jnp.arange(D)
    hsel = (c // DH)[:, None] == jnp.arange(width)[None, :]
    vals = a.reshape(D)[:, None]
    return jnp.where(hsel, vals, 0.0).astype(jnp.float32)


@jax.jit
def kernel(x_ab, x_ag, pe_ab, pe_ag, params, edge_index_abag, edge_index_agab,
           edge_index_abab, edge_index_agag):
    src_all = jnp.concatenate([
        edge_index_abag[0], edge_index_agab[0] + N_AB,
        edge_index_abab[0], edge_index_agag[0] + N_AB]).astype(jnp.int32)
    dst_all = jnp.concatenate([
        edge_index_abag[1] + N_AB, edge_index_agab[1],
        edge_index_abab[1], edge_index_agag[1] + N_AB]).astype(jnp.int32)
    ix = _prep_indices(src_all, dst_all)

    pe = jnp.concatenate([pe_ab, pe_ag], 0)
    zeros_pe = jnp.zeros_like(pe)
    x = jnp.concatenate([x_ab, x_ag], 0)

    for j in range(BLOCKS):
        for l in range(GAT_LAYERS):
            x = _gat_layer(x, pe if l == 0 else zeros_pe,
                           params["gat"][j][l], ix)
        x = _gt_layer(x, params["gt"][j], ix)

    ap = params["all_edge"][BLOCKS - 1]
    a_t, b_t = _ab_dense(x, ap["W"][:D], ap["W"][D:])
    s_cat = jnp.concatenate([
        edge_index_abag[0], edge_index_agab[0] + N_AB,
        edge_index_abab[0], edge_index_agag[0] + N_AB]).astype(jnp.int32)
    d_cat = jnp.concatenate([
        edge_index_abag[1] + N_AB, edge_index_agab[1],
        edge_index_abab[1], edge_index_agag[1] + N_AB]).astype(jnp.int32)
    ga = _sc_gather(a_t, s_cat.reshape(E4 // _CH, _CH), 256)
    gb = _sc_gather(b_t, d_cat.reshape(E4 // _CH, _CH), 256)
    ecat = _relu_add(ga, gb, ap["b"])
    y_abag = ecat[0:E]
    y_agab = ecat[E:2 * E]
    y_abab = ecat[2 * E:3 * E]
    y_agag = ecat[3 * E:4 * E]
    return (x[:N_AB], x[N_AB:], y_abag, y_agab, y_abab, y_agag)


# speedup vs baseline: 15.0317x; 1.3227x over previous
import functools

import jax
import jax.numpy as jnp
import numpy as np
from jax import lax
from jax.experimental import pallas as pl
from jax.experimental.pallas import tpu as pltpu
from jax.experimental.pallas import tpu_sc as plsc

N_AB = 5000
N_AG = 5000
N = N_AB + N_AG
E = 40000
E4 = 4 * E
D = 256
H = 8
DH = D // H
BLOCKS = 3
GAT_LAYERS = 2

B = 256
W = B + 8
NB = E4 // B
UB = 5
NB2 = NB // UB
RMAXP = 81 * 128
NP = RMAXP
NEG = -1e30


_NW = 32
_CH = 128


def _sc_gather(table, idx2, dt):
    nc = idx2.shape[0]
    tpw = -(-nc // _NW)

    def body(table_hbm, idx_hbm, out_hbm, idx_v, rows_v, sem):
        wid = lax.axis_index("s") * 2 + lax.axis_index("c")

        def step(t, carry):
            cid = t * _NW + wid

            @pl.when(cid < nc)
            def _():
                pltpu.sync_copy(idx_hbm.at[cid], idx_v)
                pltpu.async_copy(table_hbm.at[idx_v], rows_v, sem).wait()
                pltpu.sync_copy(rows_v, out_hbm.at[pl.ds(cid * _CH, _CH)])

            return carry

        lax.fori_loop(0, tpw, step, 0)

    return pl.kernel(
        body,
        out_type=jax.ShapeDtypeStruct((nc * _CH, dt), jnp.float32),
        mesh=plsc.VectorSubcoreMesh(core_axis_name="c", subcore_axis_name="s"),
        scratch_types=[
            pltpu.VMEM((_CH,), jnp.int32),
            pltpu.VMEM((_CH, dt), jnp.float32),
            pltpu.SemaphoreType.DMA,
        ],
    )(table, idx2)


BN = 400
NDB = N // BN


def _full(shape):
    return pl.BlockSpec(shape, lambda *_: tuple(0 for _ in shape))


def _gat_dense(x, pe, Wm, a_dst_mat128):
    def body(x_ref, pe_ref, w_ref, adm_ref, g_ref, t_ref):
        xb = x_ref[...] + pe_ref[...]
        h = jnp.dot(xb, w_ref[...], preferred_element_type=jnp.float32)
        g_ref[...] = h
        t_ref[...] = jnp.dot(h, adm_ref[...],
                             preferred_element_type=jnp.float32)

    return pl.pallas_call(
        body,
        grid=(NDB,),
        in_specs=[
            pl.BlockSpec((BN, D), lambda i: (i, 0)),
            pl.BlockSpec((BN, D), lambda i: (i, 0)),
            _full((D, D)),
            _full((D, 128)),
        ],
        out_specs=[
            pl.BlockSpec((BN, D), lambda i: (i, 0)),
            pl.BlockSpec((BN, 128), lambda i: (i, 0)),
        ],
        out_shape=[
            jax.ShapeDtypeStruct((N, D), jnp.float32),
            jax.ShapeDtypeStruct((N, 128), jnp.float32),
        ],
    )(x, pe, Wm, a_dst_mat128)


def _qkv_dense(x, Wq, Wk, Wv):
    def body(x_ref, wq_ref, wk_ref, wv_ref, kv_ref, q_ref):
        xb = x_ref[...]
        kv_ref[:, 0:256] = jnp.dot(xb, wk_ref[...],
                                   preferred_element_type=jnp.float32)
        kv_ref[:, 256:512] = jnp.dot(xb, wv_ref[...],
                                     preferred_element_type=jnp.float32)
        q_ref[...] = jnp.dot(xb, wq_ref[...],
                             preferred_element_type=jnp.float32)

    return pl.pallas_call(
        body,
        grid=(NDB,),
        in_specs=[pl.BlockSpec((BN, D), lambda i: (i, 0)),
                  _full((D, D)), _full((D, D)), _full((D, D))],
        out_specs=[pl.BlockSpec((BN, 512), lambda i: (i, 0)),
                   pl.BlockSpec((BN, D), lambda i: (i, 0))],
        out_shape=[jax.ShapeDtypeStruct((N, 512), jnp.float32),
                   jax.ShapeDtypeStruct((N, D), jnp.float32)],
    )(x, Wq, Wk, Wv)


def _gt_post(x, agg, p):
    def body(x_ref, a_ref, wo_ref, w1_ref, w2_ref, v_ref, o_ref):
        ln1_s = v_ref[0:1, 0:256]
        ln1_b = v_ref[1:2, 0:256]
        ln2_s = v_ref[2:3, 0:256]
        ln2_b = v_ref[3:4, 0:256]
        b1 = v_ref[4:5, 0:512]
        b2 = v_ref[5:6, 0:256]
        xb = x_ref[...]
        a = xb + jnp.dot(a_ref[...], wo_ref[...],
                         preferred_element_type=jnp.float32)
        mu = jnp.mean(a, axis=1, keepdims=True)
        var = jnp.mean((a - mu) ** 2, axis=1, keepdims=True)
        x1 = (a - mu) / jnp.sqrt(var + 1e-5) * ln1_s + ln1_b
        f = jnp.maximum(jnp.dot(x1, w1_ref[...],
                                preferred_element_type=jnp.float32) + b1, 0.0)
        f = jnp.dot(f, w2_ref[...], preferred_element_type=jnp.float32) + b2
        a2 = x1 + f
        mu2 = jnp.mean(a2, axis=1, keepdims=True)
        var2 = jnp.mean((a2 - mu2) ** 2, axis=1, keepdims=True)
        o_ref[...] = (a2 - mu2) / jnp.sqrt(var2 + 1e-5) * ln2_s + ln2_b

    vecs = jnp.zeros((6, 512), jnp.float32)
    vecs = vecs.at[0, 0:256].set(p["ln1_s"])
    vecs = vecs.at[1, 0:256].set(p["ln1_b"])
    vecs = vecs.at[2, 0:256].set(p["ln2_s"])
    vecs = vecs.at[3, 0:256].set(p["ln2_b"])
    vecs = vecs.at[4, 0:512].set(p["b1"])
    vecs = vecs.at[5, 0:256].set(p["b2"])
    return pl.pallas_call(
        body,
        grid=(NDB,),
        in_specs=[pl.BlockSpec((BN, D), lambda i: (i, 0)),
                  pl.BlockSpec((BN, D), lambda i: (i, 0)),
                  _full((D, D)), _full((D, 2 * D)), _full((2 * D, D)),
                  _full((6, 512))],
        out_specs=pl.BlockSpec((BN, D), lambda i: (i, 0)),
        out_shape=jax.ShapeDtypeStruct((N, D), jnp.float32),
    )(x, agg, p["Wo"], p["W1"], p["W2"], vecs)


def _ab_dense(x, Wtop, Wbot):
    def body(x_ref, wt_ref, wb_ref, a_ref, b_ref):
        xb = x_ref[...]
        a_ref[...] = jnp.dot(xb, wt_ref[...],
                             preferred_element_type=jnp.float32)
        b_ref[...] = jnp.dot(xb, wb_ref[...],
                             preferred_element_type=jnp.float32)

    return pl.pallas_call(
        body,
        grid=(NDB,),
        in_specs=[pl.BlockSpec((BN, D), lambda i: (i, 0)),
                  _full((D, D)), _full((D, D))],
        out_specs=[pl.BlockSpec((BN, D), lambda i: (i, 0)),
                   pl.BlockSpec((BN, D), lambda i: (i, 0))],
        out_shape=[jax.ShapeDtypeStruct((N, D), jnp.float32),
                   jax.ShapeDtypeStruct((N, D), jnp.float32)],
    )(x, Wtop, Wbot)


def _relu_add(ga, gb, bias):
    ne = ga.shape[0]
    bn = 2000

    def body(a_ref, b_ref, v_ref, o_ref):
        o_ref[...] = jnp.maximum(a_ref[...] + b_ref[...] + v_ref[...], 0.0)

    return pl.pallas_call(
        body,
        grid=(ne // bn,),
        in_specs=[pl.BlockSpec((bn, D), lambda i: (i, 0)),
                  pl.BlockSpec((bn, D), lambda i: (i, 0)),
                  _full((1, D))],
        out_specs=pl.BlockSpec((bn, D), lambda i: (i, 0)),
        out_shape=jax.ShapeDtypeStruct((ne, D), jnp.float32),
    )(ga, gb, bias.reshape(1, D))


def _onehot_t(loc2d):
    iw = lax.broadcasted_iota(jnp.int32, (W, B), 0)
    mb = iw == loc2d
    return mb.astype(jnp.float32), mb


def _seg_bound(mt, es):
    bmax = jnp.max(es, axis=0, keepdims=True)
    e1 = jnp.exp((es - bmax) * (1.0 / 16.0))
    s1 = lax.dot_general(mt, e1, (((1,), (0,)), ((), ())),
                         preferred_element_type=jnp.float32)
    c1 = jnp.where(s1 > 0, 16.0 * jnp.log(s1), -1392.0)
    c1e = lax.dot_general(mt, c1, (((0,), (0,)), ((), ())),
                          preferred_element_type=jnp.float32)
    e2 = jnp.exp((es - bmax - c1e) * 0.5)
    s2 = lax.dot_general(mt, e2, (((1,), (0,)), ((), ())),
                         preferred_element_type=jnp.float32)
    return 2.0 * jnp.log(s2) + c1 + bmax


def _expand_mat():
    r = lax.broadcasted_iota(jnp.int32, (H, D), 0)
    c = lax.broadcasted_iota(jnp.int32, (H, D), 1)
    return (c // DH == r).astype(jnp.float32)


def _gat_pass1(ge, sdr, asm, loc3, r0a):
    def body(s_ref, g_ref, sdr_ref, asm_ref, loc_ref, m_ref, es_ref):
        i = pl.program_id(0)

        @pl.when(i == 0)
        def _():
            m_ref[...] = jnp.full((RMAXP, H), NEG, jnp.float32)

        for u in range(UB):
            r0 = pl.multiple_of(s_ref[i * UB + u], 8)
            mt, mb = _onehot_t(loc_ref[0][:, u * B:(u + 1) * B])
            sd_sl = sdr_ref[pl.ds(r0, W), 0:8]
            sd_e = lax.dot_general(mt, sd_sl, (((0,), (0,)), ((), ())),
                                   preferred_element_type=jnp.float32)
            ss = jnp.dot(g_ref[u * B:(u + 1) * B, :], asm_ref[...],
                         preferred_element_type=jnp.float32)
            es = ss + sd_e
            es = jnp.where(es >= 0, es, 0.2 * es)
            es_ref[u * B:(u + 1) * B, :] = es
            bm = _seg_bound(mt, es)
            m_ref[pl.ds(r0, W), :] = jnp.maximum(m_ref[pl.ds(r0, W), :], bm)

    grid_spec = pltpu.PrefetchScalarGridSpec(
        num_scalar_prefetch=1,
        grid=(NB2,),
        in_specs=[
            pl.BlockSpec((UB * B, D), lambda i, s: (i, 0)),
            _full((RMAXP, 128)),
            _full((D, H)),
            pl.BlockSpec((1, 1, UB * B), lambda i, s: (i, 0, 0)),
        ],
        out_specs=[_full((RMAXP, H)),
                   pl.BlockSpec((UB * B, H), lambda i, s: (i, 0))],
    )
    return pl.pallas_call(
        body, grid_spec=grid_spec,
        out_shape=[jax.ShapeDtypeStruct((RMAXP, H), jnp.float32),
                   jax.ShapeDtypeStruct((E4, H), jnp.float32)],
    )(r0a, ge, sdr, asm, loc3)


def _gat_pass2(ge, es_all, m, loc3, r0a):
    def body(s_ref, g_ref, es_ref, m_ref, loc_ref, den_ref, num_ref):
        i = pl.program_id(0)

        @pl.when(i == 0)
        def _():
            den_ref[...] = jnp.zeros((RMAXP, H), jnp.float32)
            num_ref[...] = jnp.zeros((RMAXP, D), jnp.float32)

        for u in range(UB):
            r0 = pl.multiple_of(s_ref[i * UB + u], 8)
            mt, mb = _onehot_t(loc_ref[0][:, u * B:(u + 1) * B])
            g = g_ref[u * B:(u + 1) * B, :]
            es = es_ref[u * B:(u + 1) * B, :]
            m_sl = m_ref[pl.ds(r0, W), :]
            m_e = lax.dot_general(mt, m_sl, (((0,), (0,)), ((), ())),
                                  preferred_element_type=jnp.float32)
            ex = jnp.exp(es - m_e)
            den_ref[pl.ds(r0, W), :] += lax.dot_general(
                mt, ex, (((1,), (0,)), ((), ())),
                preferred_element_type=jnp.float32)
            ex_wide = lax.dot_general(ex, _expand_mat(),
                                      (((1,), (0,)), ((), ())),
                                      preferred_element_type=jnp.float32)
            vals = g * ex_wide
            num_ref[pl.ds(r0, W), :] += lax.dot_general(
                mt, vals, (((1,), (0,)), ((), ())),
                preferred_element_type=jnp.float32)

    grid_spec = pltpu.PrefetchScalarGridSpec(
        num_scalar_prefetch=1,
        grid=(NB2,),
        in_specs=[
            pl.BlockSpec((UB * B, D), lambda i, s: (i, 0)),
            pl.BlockSpec((UB * B, H), lambda i, s: (i, 0)),
            _full((RMAXP, H)),
            pl.BlockSpec((1, 1, UB * B), lambda i, s: (i, 0, 0)),
        ],
        out_specs=[_full((RMAXP, H)), _full((RMAXP, D))],
    )
    return pl.pallas_call(
        body, grid_spec=grid_spec,
        out_shape=[jax.ShapeDtypeStruct((RMAXP, H), jnp.float32),
                   jax.ShapeDtypeStruct((RMAXP, D), jnp.float32)],
    )(r0a, ge, es_all, m, loc3)


def _gt_pass1(gkv, qr, loc3, r0a):
    inv = float(1.0 / np.sqrt(DH))

    def body(s_ref, k_ref, qr_ref, loc_ref, m_ref, sc_ref):
        i = pl.program_id(0)

        @pl.when(i == 0)
        def _():
            m_ref[...] = jnp.full((RMAXP, H), NEG, jnp.float32)

        for u in range(UB):
            r0 = pl.multiple_of(s_ref[i * UB + u], 8)
            mt, mb = _onehot_t(loc_ref[0][:, u * B:(u + 1) * B])
            q_sl = qr_ref[pl.ds(r0, W), :]
            q_e = lax.dot_general(mt, q_sl, (((0,), (0,)), ((), ())),
                                  preferred_element_type=jnp.float32)
            qk = q_e * k_ref[u * B:(u + 1) * B, :]
            sc = lax.dot_general(qk, _expand_mat(), (((1,), (1,)), ((), ())),
                                 preferred_element_type=jnp.float32) * inv
            sc_ref[u * B:(u + 1) * B, :] = sc
            bm = _seg_bound(mt, sc)
            m_ref[pl.ds(r0, W), :] = jnp.maximum(m_ref[pl.ds(r0, W), :], bm)

    grid_spec = pltpu.PrefetchScalarGridSpec(
        num_scalar_prefetch=1,
        grid=(NB2,),
        in_specs=[
            pl.BlockSpec((UB * B, 256), lambda i, s: (i, 0)),
            _full((RMAXP, D)),
            pl.BlockSpec((1, 1, UB * B), lambda i, s: (i, 0, 0)),
        ],
        out_specs=[_full((RMAXP, H)),
                   pl.BlockSpec((UB * B, H), lambda i, s: (i, 0))],
    )
    return pl.pallas_call(
        body, grid_spec=grid_spec,
        out_shape=[jax.ShapeDtypeStruct((RMAXP, H), jnp.float32),
                   jax.ShapeDtypeStruct((E4, H), jnp.float32)],
    )(r0a, gkv, qr, loc3)


def _gt_pass2(gkv, sc, m, loc3, r0a):
    def body(s_ref, v_ref, sc_ref, m_ref, loc_ref, den_ref, num_ref):
        i = pl.program_id(0)

        @pl.when(i == 0)
        def _():
            den_ref[...] = jnp.zeros((RMAXP, H), jnp.float32)
            num_ref[...] = jnp.zeros((RMAXP, D), jnp.float32)

        for u in range(UB):
            r0 = pl.multiple_of(s_ref[i * UB + u], 8)
            mt, mb = _onehot_t(loc_ref[0][:, u * B:(u + 1) * B])
            m_sl = m_ref[pl.ds(r0, W), :]
            m_e = lax.dot_general(mt, m_sl, (((0,), (0,)), ((), ())),
                                  preferred_element_type=jnp.float32)
            ex = jnp.exp(sc_ref[u * B:(u + 1) * B, :] - m_e)
            den_ref[pl.ds(r0, W), :] += lax.dot_general(
                mt, ex, (((1,), (0,)), ((), ())),
                preferred_element_type=jnp.float32)
            ex_wide = lax.dot_general(ex, _expand_mat(),
                                      (((1,), (0,)), ((), ())),
                                      preferred_element_type=jnp.float32)
            vals = v_ref[u * B:(u + 1) * B, :] * ex_wide
            num_ref[pl.ds(r0, W), :] += lax.dot_general(
                mt, vals, (((1,), (0,)), ((), ())),
                preferred_element_type=jnp.float32)

    grid_spec = pltpu.PrefetchScalarGridSpec(
        num_scalar_prefetch=1,
        grid=(NB2,),
        in_specs=[
            pl.BlockSpec((UB * B, 256), lambda i, s: (i, 1)),
            pl.BlockSpec((UB * B, H), lambda i, s: (i, 0)),
            _full((RMAXP, H)),
            pl.BlockSpec((1, 1, UB * B), lambda i, s: (i, 0, 0)),
        ],
        out_specs=[_full((RMAXP, H)), _full((RMAXP, D))],
    )
    return pl.pallas_call(
        body, grid_spec=grid_spec,
        out_shape=[jax.ShapeDtypeStruct((RMAXP, H), jnp.float32),
                   jax.ShapeDtypeStruct((RMAXP, D), jnp.float32)],
    )(r0a, gkv, sc, m, loc3)


def _finalize(num, den, act):
    rb = 128

    def body(n_ref, d_ref, o_ref):
        d_wide = lax.dot_general(d_ref[...], _expand_mat(),
                                 (((1,), (0,)), ((), ())),
                                 preferred_element_type=jnp.float32)
        v = n_ref[...] / (d_wide + 1e-9)
        if act == "elu":
            v = jnp.where(v > 0, v, jnp.exp(jnp.minimum(v, 0.0)) - 1.0)
        o_ref[...] = v

    return pl.pallas_call(
        body,
        grid=(RMAXP // rb,),
        in_specs=[pl.BlockSpec((rb, D), lambda i: (i, 0)),
                  pl.BlockSpec((rb, H), lambda i: (i, 0))],
        out_specs=pl.BlockSpec((rb, D), lambda i: (i, 0)),
        out_shape=jax.ShapeDtypeStruct((RMAXP, D), jnp.float32),
    )(num, den)


def _prep_indices(src_all, dst_all):
    perm = jnp.argsort(dst_all)
    dst_s = dst_all[perm]
    src_s = src_all[perm]
    f0 = jnp.concatenate([
        jnp.ones((1,), jnp.int32),
        (dst_s[1:] != dst_s[:-1]).astype(jnp.int32)])
    rank = jnp.cumsum(f0) - 1
    nrank = rank[-1] + 1
    ar = jnp.arange(RMAXP, dtype=jnp.int32)
    node_of_rank = jnp.zeros((RMAXP,), jnp.int32).at[rank].set(dst_s)
    node_of_rank = jnp.where(ar < nrank, node_of_rank, 0)
    has_edge = jnp.zeros((N,), jnp.bool_).at[dst_s].set(True)
    rank_of_node = jnp.zeros((N,), jnp.int32).at[dst_s].set(rank)
    rank_of_node = jnp.where(has_edge, rank_of_node, RMAXP - 1)
    rank_of_node_p = jnp.concatenate(
        [rank_of_node, jnp.zeros((NP - N,), jnp.int32)])
    r0a = (rank[::B] // 8) * 8
    loc = rank - jnp.repeat(r0a, B)
    return {
        "src_s2": src_s.reshape(E4 // _CH, _CH),
        "nor2": node_of_rank.reshape(RMAXP // _CH, _CH),
        "ron2": rank_of_node_p.reshape(NP // _CH, _CH),
        "r0a": r0a.astype(jnp.int32),
        "loc3": loc.reshape(NB2, 1, UB * B).astype(jnp.int32),
    }


def _gat_layer(x, pe, lp, ix):
    asm = _head_mat(lp["a_src"], H)
    adm = _head_mat(lp["a_dst"], 128)
    th, tsd = _gat_dense(x, pe, lp["W"], adm)
    sdr = _sc_gather(tsd, ix["nor2"], 128)
    ge = _sc_gather(th, ix["src_s2"], 256)
    m, es = _gat_pass1(ge, sdr, asm, ix["loc3"], ix["r0a"])
    den, num = _gat_pass2(ge, es, m, ix["loc3"], ix["r0a"])
    fin = _finalize(num, den, "elu")
    return _sc_gather(fin, ix["ron2"], 256)[:N]


def _gt_layer(x, p, ix):
    kv, q = _qkv_dense(x, p["Wq"], p["Wk"], p["Wv"])
    qr = _sc_gather(q, ix["nor2"], 256)
    gkv = _sc_gather(kv, ix["src_s2"], 512)
    m, sc = _gt_pass1(gkv, qr, ix["loc3"], ix["r0a"])
    den, num = _gt_pass2(gkv, sc, m, ix["loc3"], ix["r0a"])
    fin = _finalize(num, den, "none")
    agg = _sc_gather(fin, ix["ron2"], 256)[:N]
    return _gt_post(x, agg, p)


def _head_mat(a, width):
    c = jnp.arange(D)
    hsel = (c // DH)[:, None] == jnp.arange(width)[None, :]
    vals = a.reshape(D)[:, None]
    return jnp.where(hsel, vals, 0.0).astype(jnp.float32)


@jax.jit
def kernel(x_ab, x_ag, pe_ab, pe_ag, params, edge_index_abag, edge_index_agab,
           edge_index_abab, edge_index_agag):
    src_all = jnp.concatenate([
        edge_index_abag[0], edge_index_agab[0] + N_AB,
        edge_index_abab[0], edge_index_agag[0] + N_AB]).astype(jnp.int32)
    dst_all = jnp.concatenate([
        edge_index_abag[1] + N_AB, edge_index_agab[1],
        edge_index_abab[1], edge_index_agag[1] + N_AB]).astype(jnp.int32)
    ix = _prep_indices(src_all, dst_all)

    pe = jnp.concatenate([pe_ab, pe_ag], 0)
    zeros_pe = jnp.zeros_like(pe)
    x = jnp.concatenate([x_ab, x_ag], 0)

    for j in range(BLOCKS):
        for l in range(GAT_LAYERS):
            x = _gat_layer(x, pe if l == 0 else zeros_pe,
                           params["gat"][j][l], ix)
        x = _gt_layer(x, params["gt"][j], ix)

    ap = params["all_edge"][BLOCKS - 1]
    a_t, b_t = _ab_dense(x, ap["W"][:D], ap["W"][D:])
    s_cat = jnp.concatenate([
        edge_index_abag[0], edge_index_agab[0] + N_AB,
        edge_index_abab[0], edge_index_agag[0] + N_AB]).astype(jnp.int32)
    d_cat = jnp.concatenate([
        edge_index_abag[1] + N_AB, edge_index_agab[1],
        edge_index_abab[1], edge_index_agag[1] + N_AB]).astype(jnp.int32)
    ga = _sc_gather(a_t, s_cat.reshape(E4 // _CH, _CH), 256)
    gb = _sc_gather(b_t, d_cat.reshape(E4 // _CH, _CH), 256)
    ecat = _relu_add(ga, gb, ap["b"])
    y_abag = ecat[0:E]
    y_agab = ecat[E:2 * E]
    y_abab = ecat[2 * E:3 * E]
    y_agag = ecat[3 * E:4 * E]
    return (x[:N_AB], x[N_AB:], y_abag, y_agab, y_abab, y_agag)
